# baseline (speedup 1.0000x reference)
import jax
import jax.numpy as jnp
from jax.experimental import pallas as pl
from jax.experimental.pallas import tpu as pltpu

N = 8192
INPUT_D = 256
HIDDEN_D = 128
OUTPUT_D = 64
ALP = 0.5
LAM = 1.0

BM = 256
HW = N // 2
P = N // BM


def _fused_kernel(
    x_ref, w1_ref, b1_ref, a0_ref, a1_ref, d0_ref, d1_ref, w2_ref, b2_ref,
    out_ref, h_scr, y1_scr, dh_scr,
):
    i = pl.program_id(0)

    @pl.when(i == 0)
    def _():
        h_scr[...] = (
            jnp.dot(x_ref[...], w1_ref[...], preferred_element_type=jnp.float32)
            + b1_ref[...]
        )

    @pl.when(i < P)
    def _():
        h0 = h_scr[pl.ds(0, HW), :]
        h1 = h_scr[pl.ds(HW, HW), :]
        ah = jnp.dot(a0_ref[...], h0, preferred_element_type=jnp.float32)
        ah += jnp.dot(a1_ref[...], h1, preferred_element_type=jnp.float32)
        dh = jnp.dot(d0_ref[...], h0, preferred_element_type=jnp.float32)
        dh += jnp.dot(d1_ref[...], h1, preferred_element_type=jnp.float32)
        rows = pl.ds(i * BM, BM)
        dh_scr[rows, :] = dh
        y1_scr[rows, :] = (1.0 - ALP) * h_scr[rows, :] + (ALP * LAM) * ah + ALP * dh

    @pl.when(i >= P)
    def _():
        j = i - P
        y0 = y1_scr[pl.ds(0, HW), :]
        y1 = y1_scr[pl.ds(HW, HW), :]
        ay = jnp.dot(a0_ref[...], y0, preferred_element_type=jnp.float32)
        ay += jnp.dot(a1_ref[...], y1, preferred_element_type=jnp.float32)
        rows = pl.ds(j * BM, BM)
        y2 = (
            (1.0 - ALP) * y1_scr[rows, :]
            + (ALP * LAM) * ay
            + ALP * dh_scr[rows, :]
        )
        z = jnp.maximum(y2, 0.0)
        out_ref[...] = (
            jnp.dot(z, w2_ref[...], preferred_element_type=jnp.float32)
            + b2_ref[...]
        )


def kernel(x, sem_adj, norm_diag, W_bef, b_bef, W_aft, b_aft):
    a_map = lambda i: (jnp.where(i < P, i, i - P), 0)
    a_map1 = lambda i: (jnp.where(i < P, i, i - P), 1)
    d_map = lambda i: (jnp.minimum(i, P - 1), 0)
    d_map1 = lambda i: (jnp.minimum(i, P - 1), 1)
    out = pl.pallas_call(
        _fused_kernel,
        grid=(2 * P,),
        in_specs=[
            pl.BlockSpec((N, INPUT_D), lambda i: (0, 0)),
            pl.BlockSpec((INPUT_D, HIDDEN_D), lambda i: (0, 0)),
            pl.BlockSpec((1, HIDDEN_D), lambda i: (0, 0)),
            pl.BlockSpec((BM, HW), a_map),
            pl.BlockSpec((BM, HW), a_map1),
            pl.BlockSpec((BM, HW), d_map),
            pl.BlockSpec((BM, HW), d_map1),
            pl.BlockSpec((HIDDEN_D, OUTPUT_D), lambda i: (0, 0)),
            pl.BlockSpec((1, OUTPUT_D), lambda i: (0, 0)),
        ],
        out_specs=pl.BlockSpec(
            (BM, OUTPUT_D), lambda i: (jnp.maximum(i - P, 0), 0)
        ),
        out_shape=jax.ShapeDtypeStruct((N, OUTPUT_D), jnp.float32),
        scratch_shapes=[
            pltpu.VMEM((N, HIDDEN_D), jnp.float32),
            pltpu.VMEM((N, HIDDEN_D), jnp.float32),
            pltpu.VMEM((N, HIDDEN_D), jnp.float32),
        ],
        compiler_params=pltpu.CompilerParams(
            dimension_semantics=("arbitrary",),
        ),
    )(
        x, W_bef, b_bef.reshape(1, HIDDEN_D), sem_adj, sem_adj,
        norm_diag, norm_diag, W_aft, b_aft.reshape(1, OUTPUT_D),
    )
    return out
